# Initial kernel scaffold; baseline (speedup 1.0000x reference)
import jax
import jax.numpy as jnp
from jax.experimental import pallas as pl


def kernel(x, x_str, edge_index, lin0_w, lin0_b, lin1_w, lin1_b, lin11_w, lin11_b, lin2_w, lin2_b, convs_w1, convs_w2, convs1_w1, convs1_w2):
    raise NotImplementedError("write your pallas kernel here")


# speedup vs baseline: 13.0507x; 13.0507x over previous
import functools
import math

import jax
import jax.numpy as jnp
from jax import lax
from jax.experimental import pallas as pl
from jax.experimental.pallas import tpu as pltpu
from jax.experimental.pallas import tpu_sc as plsc

N = 50000
E = 800000
H = 64
ALPHA = 0.9
THETA = 0.2
NUM_LAYERS = 2

NPAD = 50176
ROWS_PER_TILE = NPAD // 16
EBLK = 512
EDGES_PER_TILE = ((E // 16 + EBLK - 1) // EBLK) * EBLK
EPAD = EDGES_PER_TILE * 16
NITER = EDGES_PER_TILE // EBLK
ZROWS = 98
WB_CHUNK = 392

DEG_BLK = 512
DEG_EDGES_PER_TILE = EPAD // 32
DEG_NITER = DEG_EDGES_PER_TILE // DEG_BLK


def _zero_fill(buf, nrows):
    z = jnp.zeros((16,), jnp.float32)

    def body(i, _):
        buf[i, pl.ds(0, 16)] = z
        buf[i, pl.ds(16, 16)] = z
        return 0

    lax.fori_loop(0, nrows, body, 0)


@functools.cache
def _build_sc_kernels():
    mesh = plsc.VectorSubcoreMesh(
        core_axis_name="c", subcore_axis_name="s")

    @functools.partial(
        pl.kernel,
        out_type=[jax.ShapeDtypeStruct((NPAD,), jnp.float32)] * 2,
        mesh=mesh,
        scratch_types=[
            pltpu.VMEM((DEG_BLK,), jnp.int32),
            pltpu.VMEM((DEG_BLK,), jnp.float32),
            pltpu.VMEM((ROWS_PER_TILE,), jnp.float32),
            pltpu.VMEM_SHARED((NPAD,), jnp.float32),
        ],
    )
    def degree_kernel(colp, out0, out1, colbuf, ones, zbuf, acc):
        c = lax.axis_index("c")
        s = lax.axis_index("s")

        one = jnp.full((16,), 1.0, jnp.float32)
        z = jnp.zeros((16,), jnp.float32)

        def fill_ones(i, _):
            ones[pl.ds(i * 16, 16)] = one
            return 0

        lax.fori_loop(0, DEG_BLK // 16, fill_ones, 0)

        def fill_z(i, _):
            zbuf[pl.ds(i * 16, 16)] = z
            return 0

        lax.fori_loop(0, ROWS_PER_TILE // 16, fill_z, 0)

        pltpu.sync_copy(zbuf, acc.at[pl.ds(s * ROWS_PER_TILE, ROWS_PER_TILE)])
        plsc.subcore_barrier()

        def eb(i, _):
            base = (c * 16 + s) * DEG_EDGES_PER_TILE + i * DEG_BLK
            pltpu.sync_copy(colp.at[pl.ds(base, DEG_BLK)], colbuf)
            pltpu.sync_copy(ones, acc.at[colbuf], add=True)
            return 0

        lax.fori_loop(0, DEG_NITER, eb, 0)
        plsc.subcore_barrier()

        pltpu.sync_copy(acc.at[pl.ds(s * ROWS_PER_TILE, ROWS_PER_TILE)], zbuf)

        @pl.when(c == 0)
        def _():
            pltpu.sync_copy(zbuf, out0.at[pl.ds(s * ROWS_PER_TILE, ROWS_PER_TILE)])

        @pl.when(c == 1)
        def _():
            pltpu.sync_copy(zbuf, out1.at[pl.ds(s * ROWS_PER_TILE, ROWS_PER_TILE)])

    @functools.partial(
        pl.kernel,
        out_type=[jax.ShapeDtypeStruct((NPAD, 32), jnp.float32)] * 4,
        mesh=mesh,
        compiler_params=pltpu.CompilerParams(
            use_tc_tiling_on_sc=False, internal_scratch_in_bytes=131072),
        scratch_types=[
            pltpu.VMEM((EBLK,), jnp.int32),
            pltpu.VMEM((EBLK,), jnp.int32),
            pltpu.VMEM((EBLK, 32), jnp.float32),
            pltpu.VMEM((ZROWS, 32), jnp.float32),
            pltpu.VMEM_SHARED((NPAD, 32), jnp.float32),
        ],
    )
    def agg_kernel(t00, t01, t10, t11, rowp, colp, o00, o01, o10, o11,
                   rowbuf, colbuf, rows, zbuf, acc):
        c = lax.axis_index("c")
        s = lax.axis_index("s")

        _zero_fill(zbuf, ZROWS)

        def do_pass(table, out):
            def zcp(j, _):
                pltpu.sync_copy(
                    zbuf,
                    acc.at[pl.ds(s * ROWS_PER_TILE + j * ZROWS, ZROWS), :],
                )
                return 0

            lax.fori_loop(0, ROWS_PER_TILE // ZROWS, zcp, 0)
            plsc.subcore_barrier()

            def eb(i, _):
                base = s * EDGES_PER_TILE + i * EBLK
                pltpu.sync_copy(rowp.at[pl.ds(base, EBLK)], rowbuf)
                pltpu.sync_copy(colp.at[pl.ds(base, EBLK)], colbuf)
                pltpu.sync_copy(table.at[rowbuf], rows)
                pltpu.sync_copy(rows, acc.at[colbuf], add=True)
                return 0

            lax.fori_loop(0, NITER, eb, 0)
            plsc.subcore_barrier()

            def wb(j, _):
                off = s * ROWS_PER_TILE + j * WB_CHUNK
                pltpu.sync_copy(
                    acc.at[pl.ds(off, WB_CHUNK), :],
                    rows.at[pl.ds(0, WB_CHUNK), :],
                )
                pltpu.sync_copy(
                    rows.at[pl.ds(0, WB_CHUNK), :],
                    out.at[pl.ds(off, WB_CHUNK), :],
                )
                return 0

            lax.fori_loop(0, ROWS_PER_TILE // WB_CHUNK, wb, 0)
            plsc.subcore_barrier()

        @pl.when(c == 0)
        def _():
            do_pass(t00, o00)
            do_pass(t01, o01)

        @pl.when(c == 1)
        def _():
            do_pass(t10, o10)
            do_pass(t11, o11)

    return degree_kernel, agg_kernel


def _pad_rows(a):
    return jnp.zeros((NPAD, a.shape[1]), a.dtype).at[:N].set(a)


def kernel(x, x_str, edge_index, lin0_w, lin0_b, lin1_w, lin1_b,
           lin11_w, lin11_b, lin2_w, lin2_b,
           convs_w1, convs_w2, convs1_w1, convs1_w2):
    degree_kernel, agg_kernel = _build_sc_kernels()

    row = edge_index[0]
    col = edge_index[1]
    pad_e = EPAD - E
    ar = jnp.arange(pad_e, dtype=jnp.int32)
    rowp = jnp.concatenate([row, (ar * 7919) % N])
    colp = jnp.concatenate([col, N + ar % (NPAD - N)])

    deg0, deg1 = degree_kernel(colp)
    deg = deg0 + deg1
    dinv = jnp.where(deg > 0, lax.rsqrt(jnp.maximum(deg, 1e-12)), 0.0)

    xp = _pad_rows(x)
    xsp = _pad_rows(x_str)
    x0 = jax.nn.relu(xp @ lin0_w + lin0_b)
    x10 = jax.nn.relu(xsp @ lin11_w + lin11_b)

    eye = jnp.eye(H, dtype=jnp.float32)
    h, h1 = x0, x10
    for i in range(NUM_LAYERS):
        beta = math.log(THETA / (i + 1) + 1.0)
        P0 = (1.0 - ALPHA) * ((1.0 - beta) * eye + beta * convs_w1[i])
        Q0 = ALPHA * ((1.0 - beta) * eye + beta * convs_w2[i])
        P1 = (1.0 - ALPHA) * ((1.0 - beta) * eye + beta * convs1_w1[i])
        Q1 = ALPHA * ((1.0 - beta) * eye + beta * convs1_w2[i])

        hs = dinv[:, None] * h
        h1s = dinv[:, None] * h1
        a00, a01, a10, a11 = agg_kernel(
            hs[:, :32], hs[:, 32:], h1s[:, :32], h1s[:, 32:], rowp, colp)
        agg0 = jnp.concatenate([a00, a01], axis=1) * dinv[:, None]
        agg1 = jnp.concatenate([a10, a11], axis=1) * dinv[:, None]
        h = jax.nn.relu(agg0 @ P0 + x0 @ Q0)
        h1 = jax.nn.relu(agg1 @ P1 + x10 @ Q1)

    z = (h @ lin1_w + lin1_b)[:N]
    z1 = (h1 @ lin2_w + lin2_b)[:N]
    return (z, z1)


# speedup vs baseline: 17.4047x; 1.3336x over previous
import functools
import math

import jax
import jax.numpy as jnp
from jax import lax
from jax.experimental import pallas as pl
from jax.experimental.pallas import tpu as pltpu
from jax.experimental.pallas import tpu_sc as plsc

N = 50000
E = 800000
H = 64
ALPHA = 0.9
THETA = 0.2
NUM_LAYERS = 2

NPAD = 50176
ROWS_PER_TILE = NPAD // 16
EBLK = 352
NITER = 144
EDGES_PER_TILE = EBLK * NITER
EPAD = EDGES_PER_TILE * 16
ZROWS = 98
WB_CHUNK = 224

DEG_BLK = 352
DEG_EDGES_PER_TILE = EPAD // 32
DEG_NITER = DEG_EDGES_PER_TILE // DEG_BLK


def _zero_fill(buf, nrows):
    z = jnp.zeros((16,), jnp.float32)

    def body(i, _):
        buf[i, pl.ds(0, 16)] = z
        buf[i, pl.ds(16, 16)] = z
        return 0

    lax.fori_loop(0, nrows, body, 0)


@functools.cache
def _build_sc_kernels():
    mesh = plsc.VectorSubcoreMesh(
        core_axis_name="c", subcore_axis_name="s")

    @functools.partial(
        pl.kernel,
        out_type=[jax.ShapeDtypeStruct((NPAD,), jnp.float32)] * 2,
        mesh=mesh,
        scratch_types=[
            pltpu.VMEM((DEG_BLK,), jnp.int32),
            pltpu.VMEM((DEG_BLK,), jnp.float32),
            pltpu.VMEM((ROWS_PER_TILE,), jnp.float32),
            pltpu.VMEM_SHARED((NPAD,), jnp.float32),
        ],
    )
    def degree_kernel(colp, out0, out1, colbuf, ones, zbuf, acc):
        c = lax.axis_index("c")
        s = lax.axis_index("s")

        one = jnp.full((16,), 1.0, jnp.float32)
        z = jnp.zeros((16,), jnp.float32)

        def fill_ones(i, _):
            ones[pl.ds(i * 16, 16)] = one
            return 0

        lax.fori_loop(0, DEG_BLK // 16, fill_ones, 0)

        def fill_z(i, _):
            zbuf[pl.ds(i * 16, 16)] = z
            return 0

        lax.fori_loop(0, ROWS_PER_TILE // 16, fill_z, 0)

        pltpu.sync_copy(zbuf, acc.at[pl.ds(s * ROWS_PER_TILE, ROWS_PER_TILE)])
        plsc.subcore_barrier()

        def eb(i, _):
            base = (c * 16 + s) * DEG_EDGES_PER_TILE + i * DEG_BLK
            pltpu.sync_copy(colp.at[pl.ds(base, DEG_BLK)], colbuf)
            pltpu.sync_copy(ones, acc.at[colbuf], add=True)
            return 0

        lax.fori_loop(0, DEG_NITER, eb, 0)
        plsc.subcore_barrier()

        pltpu.sync_copy(acc.at[pl.ds(s * ROWS_PER_TILE, ROWS_PER_TILE)], zbuf)

        @pl.when(c == 0)
        def _():
            pltpu.sync_copy(zbuf, out0.at[pl.ds(s * ROWS_PER_TILE, ROWS_PER_TILE)])

        @pl.when(c == 1)
        def _():
            pltpu.sync_copy(zbuf, out1.at[pl.ds(s * ROWS_PER_TILE, ROWS_PER_TILE)])

    @functools.partial(
        pl.kernel,
        out_type=[jax.ShapeDtypeStruct((NPAD, 32), jnp.float32)] * 4,
        mesh=mesh,
        compiler_params=pltpu.CompilerParams(
            use_tc_tiling_on_sc=False, internal_scratch_in_bytes=131072),
        scratch_types=[
            pltpu.VMEM((4, EBLK), jnp.int32),
            pltpu.VMEM((4, EBLK), jnp.int32),
            pltpu.VMEM((2, EBLK, 32), jnp.float32),
            pltpu.VMEM((ZROWS, 32), jnp.float32),
            pltpu.VMEM_SHARED((NPAD, 32), jnp.float32),
            [pltpu.SemaphoreType.DMA] * 4,
            [pltpu.SemaphoreType.DMA] * 4,
            pltpu.SemaphoreType.DMA,
            [pltpu.SemaphoreType.DMA] * 2,
        ],
    )
    def agg_kernel(t00, t01, t10, t11, rowp, colp, o00, o01, o10, o11,
                   rowbuf, colbuf, rows, zbuf, acc,
                   sem_ir, sem_ic, sem_g, sem_s):
        c = lax.axis_index("c")
        s = lax.axis_index("s")

        _zero_fill(zbuf, ZROWS)

        def idx_start(i_blk, slot):
            base = s * EDGES_PER_TILE + i_blk * EBLK
            pltpu.async_copy(
                rowp.at[pl.ds(base, EBLK)], rowbuf.at[slot], sem_ir[slot])
            pltpu.async_copy(
                colp.at[pl.ds(base, EBLK)], colbuf.at[slot], sem_ic[slot])

        def idx_wait(i_blk, slot):
            base = s * EDGES_PER_TILE + i_blk * EBLK
            pltpu.make_async_copy(
                rowp.at[pl.ds(base, EBLK)], rowbuf.at[slot],
                sem_ir[slot]).wait()
            pltpu.make_async_copy(
                colp.at[pl.ds(base, EBLK)], colbuf.at[slot],
                sem_ic[slot]).wait()

        def do_pass(table, out):
            def zcp(j, _):
                pltpu.sync_copy(
                    zbuf,
                    acc.at[pl.ds(s * ROWS_PER_TILE + j * ZROWS, ZROWS), :],
                )
                return 0

            lax.fori_loop(0, ROWS_PER_TILE // ZROWS, zcp, 0)
            plsc.subcore_barrier()

            idx_start(0, 0)
            idx_start(1, 1)

            def quad(it, _):
                for j in range(4):
                    islot = j
                    rslot = j % 2
                    base_i = it * 4 + j

                    def drain():
                        pltpu.make_async_copy(
                            rows.at[rslot],
                            acc.at[colbuf.at[islot]],
                            sem_s[rslot]).wait()

                    if j >= 2:
                        drain()
                    else:
                        @pl.when(it > 0)
                        def _():
                            drain()

                    idx_wait(base_i, islot)

                    @pl.when(base_i + 2 < NITER)
                    def _():
                        idx_start(base_i + 2, (j + 2) % 4)

                    pltpu.async_copy(
                        table.at[rowbuf.at[islot]], rows.at[rslot],
                        sem_g).wait()

                    pltpu.async_copy(
                        rows.at[rslot], acc.at[colbuf.at[islot]],
                        sem_s[rslot], add=True)
                return 0

            lax.fori_loop(0, NITER // 4, quad, 0)

            for rslot, islot in ((0, 2), (1, 3)):
                pltpu.make_async_copy(
                    rows.at[rslot], acc.at[colbuf.at[islot]],
                    sem_s[rslot]).wait()
            plsc.subcore_barrier()

            def wb(j, _):
                off = s * ROWS_PER_TILE + j * WB_CHUNK
                pltpu.sync_copy(
                    acc.at[pl.ds(off, WB_CHUNK), :],
                    rows.at[0, pl.ds(0, WB_CHUNK), :],
                )
                pltpu.sync_copy(
                    rows.at[0, pl.ds(0, WB_CHUNK), :],
                    out.at[pl.ds(off, WB_CHUNK), :],
                )
                return 0

            lax.fori_loop(0, ROWS_PER_TILE // WB_CHUNK, wb, 0)
            plsc.subcore_barrier()

        @pl.when(c == 0)
        def _():
            do_pass(t00, o00)
            do_pass(t01, o01)

        @pl.when(c == 1)
        def _():
            do_pass(t10, o10)
            do_pass(t11, o11)

    return degree_kernel, agg_kernel


def _pad_rows(a):
    return jnp.zeros((NPAD, a.shape[1]), a.dtype).at[:N].set(a)


def kernel(x, x_str, edge_index, lin0_w, lin0_b, lin1_w, lin1_b,
           lin11_w, lin11_b, lin2_w, lin2_b,
           convs_w1, convs_w2, convs1_w1, convs1_w2):
    degree_kernel, agg_kernel = _build_sc_kernels()

    row = edge_index[0]
    col = edge_index[1]
    pad_e = EPAD - E
    ar = jnp.arange(pad_e, dtype=jnp.int32)
    rowp = jnp.concatenate([row, (ar * 7919) % N])
    colp = jnp.concatenate([col, N + ar % (NPAD - N)])

    deg0, deg1 = degree_kernel(colp)
    deg = deg0 + deg1
    dinv = jnp.where(deg > 0, lax.rsqrt(jnp.maximum(deg, 1e-12)), 0.0)

    xp = _pad_rows(x)
    xsp = _pad_rows(x_str)
    x0 = jax.nn.relu(xp @ lin0_w + lin0_b)
    x10 = jax.nn.relu(xsp @ lin11_w + lin11_b)

    eye = jnp.eye(H, dtype=jnp.float32)
    h, h1 = x0, x10
    for i in range(NUM_LAYERS):
        beta = math.log(THETA / (i + 1) + 1.0)
        P0 = (1.0 - ALPHA) * ((1.0 - beta) * eye + beta * convs_w1[i])
        Q0 = ALPHA * ((1.0 - beta) * eye + beta * convs_w2[i])
        P1 = (1.0 - ALPHA) * ((1.0 - beta) * eye + beta * convs1_w1[i])
        Q1 = ALPHA * ((1.0 - beta) * eye + beta * convs1_w2[i])

        hs = dinv[:, None] * h
        h1s = dinv[:, None] * h1
        a00, a01, a10, a11 = agg_kernel(
            hs[:, :32], hs[:, 32:], h1s[:, :32], h1s[:, 32:], rowp, colp)
        agg0 = jnp.concatenate([a00, a01], axis=1) * dinv[:, None]
        agg1 = jnp.concatenate([a10, a11], axis=1) * dinv[:, None]
        h = jax.nn.relu(agg0 @ P0 + x0 @ Q0)
        h1 = jax.nn.relu(agg1 @ P1 + x10 @ Q1)

    z = (h @ lin1_w + lin1_b)[:N]
    z1 = (h1 @ lin2_w + lin2_b)[:N]
    return (z, z1)
